# baseline (speedup 1.0000x reference)
import jax
import jax.numpy as jnp
from jax.experimental import pallas as pl
from jax.experimental.pallas import tpu as pltpu


N = 4096
BM = 256
NBLK = N // BM


def _body(a_ref, x_ref, w0_ref, b0_ref, w1_ref, b1_ref, out_ref, xw0_ref, p_ref):
    i = pl.program_id(0)

    @pl.when(i == 0)
    def _():
        xw0_ref[...] = jnp.dot(
            x_ref[...], w0_ref[...], preferred_element_type=jnp.float32
        ).astype(jnp.bfloat16)

    a = a_ref[0].astype(jnp.bfloat16)

    @pl.when(i < NBLK)
    def _():
        h = jnp.dot(a, xw0_ref[...], preferred_element_type=jnp.float32)
        h = jnp.maximum(h + b0_ref[...], 0.0)
        pb = jnp.dot(h, w1_ref[...], preferred_element_type=jnp.float32)
        p_ref[pl.ds(i * BM, BM), :] = pb.astype(jnp.bfloat16)
        out_ref[...] = pb

    @pl.when(i >= NBLK)
    def _():
        g = jnp.dot(a, p_ref[...], preferred_element_type=jnp.float32)
        g = g + b1_ref[...]
        m = jnp.max(g, axis=-1, keepdims=True)
        s = g - m
        lse = jnp.log(jnp.sum(jnp.exp(s), axis=-1, keepdims=True))
        out_ref[...] = s - lse


@jax.jit
def kernel(x, adjs, W0, b0, W1, b1):
    fin = x.shape[1]
    h_dim = W0.shape[1]
    fout = W1.shape[1]
    b0r = b0.reshape(1, h_dim)
    b1r = b1.reshape(1, fout)

    out = pl.pallas_call(
        _body,
        grid=(2 * NBLK,),
        out_shape=jax.ShapeDtypeStruct((N, fout), jnp.float32),
        in_specs=[
            pl.BlockSpec((1, BM, N), lambda i: (i // NBLK, i % NBLK, 0)),
            pl.BlockSpec((N, fin), lambda i: (0, 0)),
            pl.BlockSpec((fin, h_dim), lambda i: (0, 0)),
            pl.BlockSpec((1, h_dim), lambda i: (0, 0)),
            pl.BlockSpec((h_dim, fout), lambda i: (0, 0)),
            pl.BlockSpec((1, fout), lambda i: (0, 0)),
        ],
        out_specs=pl.BlockSpec((BM, fout), lambda i: (i % NBLK, 0)),
        scratch_shapes=[
            pltpu.VMEM((N, h_dim), jnp.bfloat16),
            pltpu.VMEM((N, fout), jnp.bfloat16),
        ],
    )(adjs, x, W0, b0r, W1, b1r)

    return out


# speedup vs baseline: 1.2149x; 1.2149x over previous
import jax
import jax.numpy as jnp
from jax.experimental import pallas as pl
from jax.experimental.pallas import tpu as pltpu


N = 4096
BM = 512
NBLK = N // BM


def _body(a_ref, x_ref, w0_ref, b0_ref, w1_ref, b1_ref, out_ref, xw0_ref, p_ref):
    i = pl.program_id(0)

    @pl.when(i == 0)
    def _():
        xw0_ref[...] = jnp.dot(
            x_ref[...], w0_ref[...], preferred_element_type=jnp.float32
        ).astype(jnp.bfloat16)

    a = a_ref[0].astype(jnp.bfloat16)

    @pl.when(i < NBLK)
    def _():
        h = jnp.dot(a, xw0_ref[...], preferred_element_type=jnp.float32)
        h = jnp.maximum(h + b0_ref[...], 0.0)
        pb = jnp.dot(h, w1_ref[...], preferred_element_type=jnp.float32)
        p_ref[pl.ds(i * BM, BM), :] = pb.astype(jnp.bfloat16)
        out_ref[...] = pb

    @pl.when(i >= NBLK)
    def _():
        g = jnp.dot(a, p_ref[...], preferred_element_type=jnp.float32)
        g = g + b1_ref[...]
        m = jnp.max(g, axis=-1, keepdims=True)
        s = g - m
        lse = jnp.log(jnp.sum(jnp.exp(s), axis=-1, keepdims=True))
        out_ref[...] = s - lse


@jax.jit
def kernel(x, adjs, W0, b0, W1, b1):
    fin = x.shape[1]
    h_dim = W0.shape[1]
    fout = W1.shape[1]
    b0r = b0.reshape(1, h_dim)
    b1r = b1.reshape(1, fout)

    out = pl.pallas_call(
        _body,
        grid=(2 * NBLK,),
        out_shape=jax.ShapeDtypeStruct((N, fout), jnp.float32),
        in_specs=[
            pl.BlockSpec((1, BM, N), lambda i: (i // NBLK, i % NBLK, 0)),
            pl.BlockSpec((N, fin), lambda i: (0, 0)),
            pl.BlockSpec((fin, h_dim), lambda i: (0, 0)),
            pl.BlockSpec((1, h_dim), lambda i: (0, 0)),
            pl.BlockSpec((h_dim, fout), lambda i: (0, 0)),
            pl.BlockSpec((1, fout), lambda i: (0, 0)),
        ],
        out_specs=pl.BlockSpec((BM, fout), lambda i: (i % NBLK, 0)),
        scratch_shapes=[
            pltpu.VMEM((N, h_dim), jnp.bfloat16),
            pltpu.VMEM((N, fout), jnp.bfloat16),
        ],
    )(adjs, x, W0, b0r, W1, b1r)

    return out


# speedup vs baseline: 1.2582x; 1.0357x over previous
import jax
import jax.numpy as jnp
from jax.experimental import pallas as pl
from jax.experimental.pallas import tpu as pltpu


N = 4096
BM = 1024
NBLK = N // BM


def _body(a_ref, x_ref, w0_ref, b0_ref, w1_ref, b1_ref, out_ref, xw0_ref, p_ref):
    i = pl.program_id(0)

    @pl.when(i == 0)
    def _():
        xw0_ref[...] = jnp.dot(
            x_ref[...], w0_ref[...], preferred_element_type=jnp.float32
        ).astype(jnp.bfloat16)

    a = a_ref[0].astype(jnp.bfloat16)

    @pl.when(i < NBLK)
    def _():
        h = jnp.dot(a, xw0_ref[...], preferred_element_type=jnp.float32)
        h = jnp.maximum(h + b0_ref[...], 0.0)
        pb = jnp.dot(h, w1_ref[...], preferred_element_type=jnp.float32)
        p_ref[pl.ds(i * BM, BM), :] = pb.astype(jnp.bfloat16)
        out_ref[...] = pb

    @pl.when(i >= NBLK)
    def _():
        g = jnp.dot(a, p_ref[...], preferred_element_type=jnp.float32)
        g = g + b1_ref[...]
        m = jnp.max(g, axis=-1, keepdims=True)
        s = g - m
        lse = jnp.log(jnp.sum(jnp.exp(s), axis=-1, keepdims=True))
        out_ref[...] = s - lse


@jax.jit
def kernel(x, adjs, W0, b0, W1, b1):
    fin = x.shape[1]
    h_dim = W0.shape[1]
    fout = W1.shape[1]
    b0r = b0.reshape(1, h_dim)
    b1r = b1.reshape(1, fout)

    out = pl.pallas_call(
        _body,
        grid=(2 * NBLK,),
        out_shape=jax.ShapeDtypeStruct((N, fout), jnp.float32),
        in_specs=[
            pl.BlockSpec((1, BM, N), lambda i: (i // NBLK, i % NBLK, 0)),
            pl.BlockSpec((N, fin), lambda i: (0, 0)),
            pl.BlockSpec((fin, h_dim), lambda i: (0, 0)),
            pl.BlockSpec((1, h_dim), lambda i: (0, 0)),
            pl.BlockSpec((h_dim, fout), lambda i: (0, 0)),
            pl.BlockSpec((1, fout), lambda i: (0, 0)),
        ],
        out_specs=pl.BlockSpec((BM, fout), lambda i: (i % NBLK, 0)),
        scratch_shapes=[
            pltpu.VMEM((N, h_dim), jnp.bfloat16),
            pltpu.VMEM((N, fout), jnp.bfloat16),
        ],
    )(adjs, x, W0, b0r, W1, b1r)

    return out
